# baseline (speedup 1.0000x reference)
import functools

import jax
import jax.numpy as jnp
from jax import lax
from jax.experimental import pallas as pl
from jax.experimental.pallas import tpu as pltpu
from jax.experimental.pallas import tpu_sc as plsc

N_NODES = 10000
N_EDGES = 320000
HID = 128
NC = 2
NS = 16
NW = NC * NS
EPW = N_EDGES // NW
CH = 80
NCHUNK = EPW // CH
UR = 40
NU = N_NODES // UR
NSUB = HID // 16


def _mm2_body(h_ref, wn_ref, ws_ref, b_ref, xn_ref, hs_ref):
    hb = h_ref[...]
    xn_ref[...] = jnp.dot(hb, wn_ref[...], preferred_element_type=jnp.float32)
    hs_ref[...] = jnp.dot(hb, ws_ref[...],
                          preferred_element_type=jnp.float32) + b_ref[...]


def _mm2(h, wn, ws, b2d, bm):
    m, k = h.shape
    n = wn.shape[1]
    return pl.pallas_call(
        _mm2_body,
        grid=(m // bm,),
        in_specs=[pl.BlockSpec((bm, k), lambda i: (i, 0)),
                  pl.BlockSpec((k, n), lambda i: (0, 0)),
                  pl.BlockSpec((k, n), lambda i: (0, 0)),
                  pl.BlockSpec((1, n), lambda i: (0, 0))],
        out_specs=[pl.BlockSpec((bm, n), lambda i: (i, 0)),
                   pl.BlockSpec((bm, n), lambda i: (i, 0))],
        out_shape=[jax.ShapeDtypeStruct((m, n), jnp.float32),
                   jax.ShapeDtypeStruct((m, n), jnp.float32)],
    )(h, wn, ws, b2d)


def _mm_body(x_ref, w_ref, o_ref):
    o_ref[...] = jnp.dot(x_ref[...], w_ref[...],
                         preferred_element_type=jnp.float32)


def _matmul(xx, ww, bm):
    m, k = xx.shape
    _, n = ww.shape
    return pl.pallas_call(
        _mm_body,
        grid=(m // bm,),
        in_specs=[pl.BlockSpec((bm, k), lambda i: (i, 0)),
                  pl.BlockSpec((k, n), lambda i: (0, 0))],
        out_specs=pl.BlockSpec((bm, n), lambda i: (i, 0)),
        out_shape=jax.ShapeDtypeStruct((m, n), jnp.float32),
    )(xx, ww)


def _upd_body(hs_ref, agg_ref, o_ref):
    o_ref[...] = jnp.maximum(hs_ref[...] + agg_ref[0] + agg_ref[1], 0.0)


def _update(hs, agg, bm):
    m, n = hs.shape
    return pl.pallas_call(
        _upd_body,
        grid=(m // bm,),
        in_specs=[pl.BlockSpec((bm, n), lambda i: (i, 0)),
                  pl.BlockSpec((2, bm, n), lambda i: (0, i, 0))],
        out_specs=pl.BlockSpec((bm, n), lambda i: (i, 0)),
        out_shape=jax.ShapeDtypeStruct((m, n), jnp.float32),
    )(hs, agg)


_MESH = plsc.VectorSubcoreMesh(core_axis_name="c", subcore_axis_name="s")


@functools.partial(
    pl.kernel,
    mesh=_MESH,
    out_type=jax.ShapeDtypeStruct((NC, N_NODES, HID), jnp.float32),
    scratch_types=[
        pltpu.VMEM((CH,), jnp.int32),
        pltpu.VMEM((CH,), jnp.int32),
        pltpu.VMEM((CH,), jnp.int32),
        pltpu.VMEM((CH,), jnp.int32),
        pltpu.VMEM((CH,), jnp.int32),
        pltpu.VMEM((CH,), jnp.int32),
        pltpu.VMEM((CH,), jnp.int32),
        pltpu.VMEM((CH,), jnp.int32),
        pltpu.VMEM((CH, HID), jnp.float32),
        pltpu.VMEM((CH, HID), jnp.float32),
        pltpu.VMEM((CH, HID), jnp.float32),
        pltpu.VMEM((CH, HID), jnp.float32),
        pltpu.VMEM((UR, HID), jnp.float32),
        pltpu.VMEM_SHARED((N_NODES, HID), jnp.float32),
        pltpu.SemaphoreType.DMA,
        pltpu.SemaphoreType.DMA,
        pltpu.SemaphoreType.DMA,
        pltpu.SemaphoreType.DMA,
        pltpu.SemaphoreType.DMA,
        pltpu.SemaphoreType.DMA,
        pltpu.SemaphoreType.DMA,
        pltpu.SemaphoreType.DMA,
        pltpu.SemaphoreType.DMA,
        pltpu.SemaphoreType.DMA,
        pltpu.SemaphoreType.DMA,
    ],
)
def _sc_edges(xn, ea, src, dst, out,
              src_v0, src_v1, src_v2, src_v3,
              dst_v0, dst_v1, dst_v2, dst_v3,
              rows_v0, rows_v1, msg_v0, msg_v1, zero_v, agg_sh,
              isem0, isem1, isem2, isem3,
              gsem0, gsem1, esem0, esem1, ssem0, ssem1, usem):
    c = lax.axis_index("c")
    s = lax.axis_index("s")
    wid = c * NS + s
    src_vs = [src_v0, src_v1, src_v2, src_v3]
    dst_vs = [dst_v0, dst_v1, dst_v2, dst_v3]
    rows_vs = [rows_v0, rows_v1]
    msg_vs = [msg_v0, msg_v1]
    isems = [isem0, isem1, isem2, isem3]
    gsems = [gsem0, gsem1]
    esems = [esem0, esem1]
    ssems = [ssem0, ssem1]

    def zfill(i, carry):
        for h in range(NSUB):
            zero_v[i, pl.ds(h * 16, 16)] = jnp.zeros((16,), jnp.float32)
        return carry

    lax.fori_loop(0, UR, zfill, 0)
    nunits = (NU - s + NS - 1) // NS

    def zunit(j, carry):
        u = s + NS * j
        pltpu.async_copy(zero_v, agg_sh.at[pl.ds(u * UR, UR), :], usem)
        return carry

    lax.fori_loop(0, nunits, zunit, 0)

    def zdrain(j, carry):
        pltpu.make_async_copy(zero_v, agg_sh.at[pl.ds(0, UR), :], usem).wait()
        return carry

    lax.fori_loop(0, nunits, zdrain, 0)
    plsc.subcore_barrier()

    def fetch_idx(i, ch):
        base = wid * EPW + ch * CH
        pltpu.async_copy(src.at[pl.ds(base, CH)], src_vs[i], isems[i])
        pltpu.async_copy(dst.at[pl.ds(base, CH)], dst_vs[i], isems[i])

    def fetch_data(b, i, ws, ch, first):
        base = wid * EPW + ch * CH
        pltpu.make_async_copy(src.at[pl.ds(0, CH)], src_vs[i],
                              isems[i]).wait()
        pltpu.make_async_copy(dst.at[pl.ds(0, CH)], dst_vs[i],
                              isems[i]).wait()
        if not first:
            pltpu.make_async_copy(msg_vs[b], agg_sh.at[dst_vs[ws]],
                                  ssems[b]).wait()
        pltpu.async_copy(xn.at[src_vs[i]], rows_vs[b], gsems[b])
        pltpu.async_copy(ea.at[pl.ds(base, CH), :], msg_vs[b], esems[b])

    def consume(b, i):
        pltpu.make_async_copy(xn.at[src_vs[i]], rows_vs[b], gsems[b]).wait()
        pltpu.make_async_copy(ea.at[pl.ds(0, CH), :], msg_vs[b],
                              esems[b]).wait()

        def _edge(e):
            for h in range(NSUB):
                sl = pl.ds(h * 16, 16)
                msg_vs[b][e, sl] = jnp.maximum(
                    rows_vs[b][e, sl] + msg_vs[b][e, sl], 0.0)

        plsc.parallel_loop(0, CH, unroll=4)(_edge)

        pltpu.async_copy(msg_vs[b], agg_sh.at[dst_vs[i]], ssems[b], add=True)

    fetch_idx(0, 0)
    fetch_idx(1, 1)
    fetch_idx(2, 2)
    fetch_data(0, 0, 0, 0, True)
    fetch_data(1, 1, 0, 1, True)
    fetch_idx(3, 3)
    consume(0, 0)
    fetch_data(0, 2, 0, 2, False)
    fetch_idx(0, 4)
    consume(1, 1)
    fetch_data(1, 3, 1, 3, False)
    fetch_idx(1, 5)
    consume(0, 2)
    fetch_data(0, 0, 2, 4, False)
    fetch_idx(2, 6)
    consume(1, 3)

    def quad(j, carry):
        for t in range(4):
            k = 4 * j + 4 + t
            fetch_data((t + 1) % 2, (t + 1) % 4, (t + 3) % 4, k + 1, False)
            fetch_idx((t + 3) % 4, jnp.minimum(k + 3, NCHUNK - 1))
            consume(t % 2, t % 4)
        return carry

    lax.fori_loop(0, (NCHUNK - 5) // 4, quad, 0)

    consume(0, 0)
    pltpu.make_async_copy(msg_vs[1], agg_sh.at[dst_vs[3]], ssems[1]).wait()
    pltpu.make_async_copy(msg_vs[0], agg_sh.at[dst_vs[0]], ssems[0]).wait()
    for i in (1, 2):
        pltpu.make_async_copy(src.at[pl.ds(0, CH)], src_vs[i],
                              isems[i]).wait()
        pltpu.make_async_copy(dst.at[pl.ds(0, CH)], dst_vs[i],
                              isems[i]).wait()

    plsc.subcore_barrier()

    def wunit(j, carry):
        u = s + NS * j
        rows = pl.ds(u * UR, UR)
        pltpu.async_copy(agg_sh.at[rows, :], out.at[c, rows, :], usem)
        return carry

    lax.fori_loop(0, nunits, wunit, 0)

    def wdrain(j, carry):
        pltpu.make_async_copy(agg_sh.at[pl.ds(0, UR), :],
                              out.at[c, pl.ds(0, UR), :], usem).wait()
        return carry

    lax.fori_loop(0, nunits, wdrain, 0)


def kernel(x, edge_index, edge_attr, Wn1, We1, Ws1, b1, Wn2, We2, Ws2, b2,
           Wn3, We3, Ws3, b3, Wn4, We4, Ws4, b4):
    src = edge_index[0]
    dst = edge_index[1]
    layers = [(Wn1, We1, Ws1, b1), (Wn2, We2, Ws2, b2),
              (Wn3, We3, Ws3, b3), (Wn4, We4, Ws4, b4)]
    eas = [_matmul(edge_attr, We, 2000) for _, We, _, _ in layers]
    h = x
    for (Wn, _, Ws, b), ea in zip(layers, eas):
        xn, hs = _mm2(h, Wn, Ws, b.reshape(1, -1), 400)
        agg = _sc_edges(xn, ea, src, dst)
        h = _update(hs, agg, 400)
    return h


# speedup vs baseline: 1.0273x; 1.0273x over previous
import functools

import jax
import jax.numpy as jnp
from jax import lax
from jax.experimental import pallas as pl
from jax.experimental.pallas import tpu as pltpu
from jax.experimental.pallas import tpu_sc as plsc

N_NODES = 10000
N_EDGES = 320000
HID = 128
NC = 2
NS = 16
NW = NC * NS
EPW = N_EDGES // NW
CH = 80
NCHUNK = EPW // CH
UR = 40
NU = N_NODES // UR
NSUB = HID // 16


def _mm2_body(h_ref, wn_ref, ws_ref, b_ref, xn_ref, hs_ref):
    hb = h_ref[...]
    xn_ref[...] = jnp.dot(hb, wn_ref[...], preferred_element_type=jnp.float32)
    hs_ref[...] = jnp.dot(hb, ws_ref[...],
                          preferred_element_type=jnp.float32) + b_ref[...]


def _mm2(h, wn, ws, b2d, bm):
    m, k = h.shape
    n = wn.shape[1]
    return pl.pallas_call(
        _mm2_body,
        grid=(m // bm,),
        in_specs=[pl.BlockSpec((bm, k), lambda i: (i, 0)),
                  pl.BlockSpec((k, n), lambda i: (0, 0)),
                  pl.BlockSpec((k, n), lambda i: (0, 0)),
                  pl.BlockSpec((1, n), lambda i: (0, 0))],
        out_specs=[pl.BlockSpec((bm, n), lambda i: (i, 0)),
                   pl.BlockSpec((bm, n), lambda i: (i, 0))],
        out_shape=[jax.ShapeDtypeStruct((m, n), jnp.float32),
                   jax.ShapeDtypeStruct((m, n), jnp.float32)],
    )(h, wn, ws, b2d)


def _mm_body(x_ref, w_ref, o_ref):
    o_ref[...] = jnp.dot(x_ref[...], w_ref[...],
                         preferred_element_type=jnp.float32)


def _matmul(xx, ww, bm):
    m, k = xx.shape
    _, n = ww.shape
    return pl.pallas_call(
        _mm_body,
        grid=(m // bm,),
        in_specs=[pl.BlockSpec((bm, k), lambda i: (i, 0)),
                  pl.BlockSpec((k, n), lambda i: (0, 0))],
        out_specs=pl.BlockSpec((bm, n), lambda i: (i, 0)),
        out_shape=jax.ShapeDtypeStruct((m, n), jnp.float32),
    )(xx, ww)


def _fused_body(hs_ref, agg_ref, wn_ref, ws_ref, b_ref, xn_ref, hsn_ref):
    h = jnp.maximum(hs_ref[...] + agg_ref[0] + agg_ref[1], 0.0)
    xn_ref[...] = jnp.dot(h, wn_ref[...], preferred_element_type=jnp.float32)
    hsn_ref[...] = jnp.dot(h, ws_ref[...],
                           preferred_element_type=jnp.float32) + b_ref[...]


def _fused(hs, agg, wn, ws, b2d, bm):
    m, n = hs.shape
    k = wn.shape[0]
    return pl.pallas_call(
        _fused_body,
        grid=(m // bm,),
        in_specs=[pl.BlockSpec((bm, n), lambda i: (i, 0)),
                  pl.BlockSpec((2, bm, n), lambda i: (0, i, 0)),
                  pl.BlockSpec((k, n), lambda i: (0, 0)),
                  pl.BlockSpec((k, n), lambda i: (0, 0)),
                  pl.BlockSpec((1, n), lambda i: (0, 0))],
        out_specs=[pl.BlockSpec((bm, n), lambda i: (i, 0)),
                   pl.BlockSpec((bm, n), lambda i: (i, 0))],
        out_shape=[jax.ShapeDtypeStruct((m, n), jnp.float32),
                   jax.ShapeDtypeStruct((m, n), jnp.float32)],
    )(hs, agg, wn, ws, b2d)


def _upd_body(hs_ref, agg_ref, o_ref):
    o_ref[...] = jnp.maximum(hs_ref[...] + agg_ref[0] + agg_ref[1], 0.0)


def _update(hs, agg, bm):
    m, n = hs.shape
    return pl.pallas_call(
        _upd_body,
        grid=(m // bm,),
        in_specs=[pl.BlockSpec((bm, n), lambda i: (i, 0)),
                  pl.BlockSpec((2, bm, n), lambda i: (0, i, 0))],
        out_specs=pl.BlockSpec((bm, n), lambda i: (i, 0)),
        out_shape=jax.ShapeDtypeStruct((m, n), jnp.float32),
    )(hs, agg)


_MESH = plsc.VectorSubcoreMesh(core_axis_name="c", subcore_axis_name="s")


@functools.partial(
    pl.kernel,
    mesh=_MESH,
    out_type=jax.ShapeDtypeStruct((NC, N_NODES, HID), jnp.float32),
    scratch_types=[
        pltpu.VMEM((CH,), jnp.int32),
        pltpu.VMEM((CH,), jnp.int32),
        pltpu.VMEM((CH,), jnp.int32),
        pltpu.VMEM((CH,), jnp.int32),
        pltpu.VMEM((CH,), jnp.int32),
        pltpu.VMEM((CH,), jnp.int32),
        pltpu.VMEM((CH,), jnp.int32),
        pltpu.VMEM((CH,), jnp.int32),
        pltpu.VMEM((CH, HID), jnp.float32),
        pltpu.VMEM((CH, HID), jnp.float32),
        pltpu.VMEM((CH, HID), jnp.float32),
        pltpu.VMEM((CH, HID), jnp.float32),
        pltpu.VMEM((UR, HID), jnp.float32),
        pltpu.VMEM_SHARED((N_NODES, HID), jnp.float32),
        pltpu.SemaphoreType.DMA,
        pltpu.SemaphoreType.DMA,
        pltpu.SemaphoreType.DMA,
        pltpu.SemaphoreType.DMA,
        pltpu.SemaphoreType.DMA,
        pltpu.SemaphoreType.DMA,
        pltpu.SemaphoreType.DMA,
        pltpu.SemaphoreType.DMA,
        pltpu.SemaphoreType.DMA,
        pltpu.SemaphoreType.DMA,
        pltpu.SemaphoreType.DMA,
    ],
)
def _sc_edges(xn, ea, src, dst, out,
              src_v0, src_v1, src_v2, src_v3,
              dst_v0, dst_v1, dst_v2, dst_v3,
              rows_v0, rows_v1, msg_v0, msg_v1, zero_v, agg_sh,
              isem0, isem1, isem2, isem3,
              gsem0, gsem1, esem0, esem1, ssem0, ssem1, usem):
    c = lax.axis_index("c")
    s = lax.axis_index("s")
    wid = c * NS + s
    src_vs = [src_v0, src_v1, src_v2, src_v3]
    dst_vs = [dst_v0, dst_v1, dst_v2, dst_v3]
    rows_vs = [rows_v0, rows_v1]
    msg_vs = [msg_v0, msg_v1]
    isems = [isem0, isem1, isem2, isem3]
    gsems = [gsem0, gsem1]
    esems = [esem0, esem1]
    ssems = [ssem0, ssem1]

    def zfill(i, carry):
        for h in range(NSUB):
            zero_v[i, pl.ds(h * 16, 16)] = jnp.zeros((16,), jnp.float32)
        return carry

    lax.fori_loop(0, UR, zfill, 0)
    nunits = (NU - s + NS - 1) // NS

    def zunit(j, carry):
        u = s + NS * j
        pltpu.async_copy(zero_v, agg_sh.at[pl.ds(u * UR, UR), :], usem)
        return carry

    lax.fori_loop(0, nunits, zunit, 0)

    def zdrain(j, carry):
        pltpu.make_async_copy(zero_v, agg_sh.at[pl.ds(0, UR), :], usem).wait()
        return carry

    lax.fori_loop(0, nunits, zdrain, 0)
    plsc.subcore_barrier()

    def fetch_idx(i, ch):
        base = wid * EPW + ch * CH
        pltpu.async_copy(src.at[pl.ds(base, CH)], src_vs[i], isems[i])
        pltpu.async_copy(dst.at[pl.ds(base, CH)], dst_vs[i], isems[i])

    def fetch_data(b, i, ws, ch, first):
        base = wid * EPW + ch * CH
        pltpu.make_async_copy(src.at[pl.ds(0, CH)], src_vs[i],
                              isems[i]).wait()
        pltpu.make_async_copy(dst.at[pl.ds(0, CH)], dst_vs[i],
                              isems[i]).wait()
        if not first:
            pltpu.make_async_copy(msg_vs[b], agg_sh.at[dst_vs[ws]],
                                  ssems[b]).wait()
        pltpu.async_copy(xn.at[src_vs[i]], rows_vs[b], gsems[b])
        pltpu.async_copy(ea.at[pl.ds(base, CH), :], msg_vs[b], esems[b])

    def consume(b, i):
        pltpu.make_async_copy(xn.at[src_vs[i]], rows_vs[b], gsems[b]).wait()
        pltpu.make_async_copy(ea.at[pl.ds(0, CH), :], msg_vs[b],
                              esems[b]).wait()

        def _edge(e):
            for h in range(NSUB):
                sl = pl.ds(h * 16, 16)
                msg_vs[b][e, sl] = jnp.maximum(
                    rows_vs[b][e, sl] + msg_vs[b][e, sl], 0.0)

        plsc.parallel_loop(0, CH, unroll=4)(_edge)

        pltpu.async_copy(msg_vs[b], agg_sh.at[dst_vs[i]], ssems[b], add=True)

    fetch_idx(0, 0)
    fetch_idx(1, 1)
    fetch_idx(2, 2)
    fetch_data(0, 0, 0, 0, True)
    fetch_data(1, 1, 0, 1, True)
    fetch_idx(3, 3)
    consume(0, 0)
    fetch_data(0, 2, 0, 2, False)
    fetch_idx(0, 4)
    consume(1, 1)
    fetch_data(1, 3, 1, 3, False)
    fetch_idx(1, 5)
    consume(0, 2)
    fetch_data(0, 0, 2, 4, False)
    fetch_idx(2, 6)
    consume(1, 3)

    def quad(j, carry):
        for t in range(4):
            k = 4 * j + 4 + t
            fetch_data((t + 1) % 2, (t + 1) % 4, (t + 3) % 4, k + 1, False)
            fetch_idx((t + 3) % 4, jnp.minimum(k + 3, NCHUNK - 1))
            consume(t % 2, t % 4)
        return carry

    lax.fori_loop(0, (NCHUNK - 5) // 4, quad, 0)

    consume(0, 0)
    pltpu.make_async_copy(msg_vs[1], agg_sh.at[dst_vs[3]], ssems[1]).wait()
    pltpu.make_async_copy(msg_vs[0], agg_sh.at[dst_vs[0]], ssems[0]).wait()
    for i in (1, 2):
        pltpu.make_async_copy(src.at[pl.ds(0, CH)], src_vs[i],
                              isems[i]).wait()
        pltpu.make_async_copy(dst.at[pl.ds(0, CH)], dst_vs[i],
                              isems[i]).wait()

    plsc.subcore_barrier()

    def wunit(j, carry):
        u = s + NS * j
        rows = pl.ds(u * UR, UR)
        pltpu.async_copy(agg_sh.at[rows, :], out.at[c, rows, :], usem)
        return carry

    lax.fori_loop(0, nunits, wunit, 0)

    def wdrain(j, carry):
        pltpu.make_async_copy(agg_sh.at[pl.ds(0, UR), :],
                              out.at[c, pl.ds(0, UR), :], usem).wait()
        return carry

    lax.fori_loop(0, nunits, wdrain, 0)


def kernel(x, edge_index, edge_attr, Wn1, We1, Ws1, b1, Wn2, We2, Ws2, b2,
           Wn3, We3, Ws3, b3, Wn4, We4, Ws4, b4):
    src = edge_index[0]
    dst = edge_index[1]
    layers = [(Wn1, We1, Ws1, b1), (Wn2, We2, Ws2, b2),
              (Wn3, We3, Ws3, b3), (Wn4, We4, Ws4, b4)]
    eas = [_matmul(edge_attr, We, 4000) for _, We, _, _ in layers]
    Wn1_, _, Ws1_, b1_ = layers[0]
    xn, hs = _mm2(x, Wn1_, Ws1_, b1_.reshape(1, -1), 400)
    agg = _sc_edges(xn, eas[0], src, dst)
    for (Wn, _, Ws, b), ea in zip(layers[1:], eas[1:]):
        xn, hs = _fused(hs, agg, Wn, Ws, b.reshape(1, -1), 400)
        agg = _sc_edges(xn, ea, src, dst)
    return _update(hs, agg, 400)


# speedup vs baseline: 1.0583x; 1.0302x over previous
import functools

import jax
import jax.numpy as jnp
from jax import lax
from jax.experimental import pallas as pl
from jax.experimental.pallas import tpu as pltpu
from jax.experimental.pallas import tpu_sc as plsc

N_NODES = 10000
N_EDGES = 320000
HID = 128
NC = 2
NS = 16
NW = NC * NS
EPW = N_EDGES // NW
CH = 80
NCHUNK = EPW // CH
UR = 40
NU = N_NODES // UR
NSUB = HID // 16


def _mm2_body(h_ref, wn_ref, ws_ref, b_ref, xn_ref, hs_ref):
    hb = h_ref[...]
    xn_ref[...] = jnp.dot(hb, wn_ref[...], preferred_element_type=jnp.float32)
    hs_ref[...] = jnp.dot(hb, ws_ref[...],
                          preferred_element_type=jnp.float32) + b_ref[...]


def _mm2(h, wn, ws, b2d, bm):
    m, k = h.shape
    n = wn.shape[1]
    return pl.pallas_call(
        _mm2_body,
        grid=(m // bm,),
        in_specs=[pl.BlockSpec((bm, k), lambda i: (i, 0)),
                  pl.BlockSpec((k, n), lambda i: (0, 0)),
                  pl.BlockSpec((k, n), lambda i: (0, 0)),
                  pl.BlockSpec((1, n), lambda i: (0, 0))],
        out_specs=[pl.BlockSpec((bm, n), lambda i: (i, 0)),
                   pl.BlockSpec((bm, n), lambda i: (i, 0))],
        out_shape=[jax.ShapeDtypeStruct((m, n), jnp.float32),
                   jax.ShapeDtypeStruct((m, n), jnp.float32)],
    )(h, wn, ws, b2d)


def _mm_body(x_ref, w_ref, o_ref):
    o_ref[...] = jnp.dot(x_ref[...], w_ref[...],
                         preferred_element_type=jnp.float32)


def _matmul(xx, ww, bm):
    m, k = xx.shape
    _, n = ww.shape
    return pl.pallas_call(
        _mm_body,
        grid=(m // bm,),
        in_specs=[pl.BlockSpec((bm, k), lambda i: (i, 0)),
                  pl.BlockSpec((k, n), lambda i: (0, 0))],
        out_specs=pl.BlockSpec((bm, n), lambda i: (i, 0)),
        out_shape=jax.ShapeDtypeStruct((m, n), jnp.float32),
    )(xx, ww)


def _fused_body(hs_ref, agg_ref, wn_ref, ws_ref, b_ref, xn_ref, hsn_ref):
    h = jnp.maximum(hs_ref[...] + agg_ref[0] + agg_ref[1], 0.0)
    xn_ref[...] = jnp.dot(h, wn_ref[...], preferred_element_type=jnp.float32)
    hsn_ref[...] = jnp.dot(h, ws_ref[...],
                           preferred_element_type=jnp.float32) + b_ref[...]


def _fused(hs, agg, wn, ws, b2d, bm):
    m, n = hs.shape
    k = wn.shape[0]
    return pl.pallas_call(
        _fused_body,
        grid=(m // bm,),
        in_specs=[pl.BlockSpec((bm, n), lambda i: (i, 0)),
                  pl.BlockSpec((2, bm, n), lambda i: (0, i, 0)),
                  pl.BlockSpec((k, n), lambda i: (0, 0)),
                  pl.BlockSpec((k, n), lambda i: (0, 0)),
                  pl.BlockSpec((1, n), lambda i: (0, 0))],
        out_specs=[pl.BlockSpec((bm, n), lambda i: (i, 0)),
                   pl.BlockSpec((bm, n), lambda i: (i, 0))],
        out_shape=[jax.ShapeDtypeStruct((m, n), jnp.float32),
                   jax.ShapeDtypeStruct((m, n), jnp.float32)],
    )(hs, agg, wn, ws, b2d)


def _upd_body(hs_ref, agg_ref, o_ref):
    o_ref[...] = jnp.maximum(hs_ref[...] + agg_ref[0] + agg_ref[1], 0.0)


def _update(hs, agg, bm):
    m, n = hs.shape
    return pl.pallas_call(
        _upd_body,
        grid=(m // bm,),
        in_specs=[pl.BlockSpec((bm, n), lambda i: (i, 0)),
                  pl.BlockSpec((2, bm, n), lambda i: (0, i, 0))],
        out_specs=pl.BlockSpec((bm, n), lambda i: (i, 0)),
        out_shape=jax.ShapeDtypeStruct((m, n), jnp.float32),
    )(hs, agg)


_MESH = plsc.VectorSubcoreMesh(core_axis_name="c", subcore_axis_name="s")


@functools.partial(
    pl.kernel,
    mesh=_MESH,
    out_type=jax.ShapeDtypeStruct((NC, N_NODES, HID), jnp.float32),
    scratch_types=[
        pltpu.VMEM((CH,), jnp.int32),
        pltpu.VMEM((CH,), jnp.int32),
        pltpu.VMEM((CH,), jnp.int32),
        pltpu.VMEM((CH,), jnp.int32),
        pltpu.VMEM((CH,), jnp.int32),
        pltpu.VMEM((CH,), jnp.int32),
        pltpu.VMEM((CH,), jnp.int32),
        pltpu.VMEM((CH,), jnp.int32),
        pltpu.VMEM((CH, HID), jnp.float32),
        pltpu.VMEM((CH, HID), jnp.float32),
        pltpu.VMEM((CH, HID), jnp.float32),
        pltpu.VMEM((CH, HID), jnp.float32),
        pltpu.VMEM((UR, HID), jnp.float32),
        pltpu.VMEM_SHARED((N_NODES, HID), jnp.float32),
        pltpu.SemaphoreType.DMA,
        pltpu.SemaphoreType.DMA,
        pltpu.SemaphoreType.DMA,
        pltpu.SemaphoreType.DMA,
        pltpu.SemaphoreType.DMA,
        pltpu.SemaphoreType.DMA,
        pltpu.SemaphoreType.DMA,
        pltpu.SemaphoreType.DMA,
        pltpu.SemaphoreType.DMA,
        pltpu.SemaphoreType.DMA,
        pltpu.SemaphoreType.DMA,
    ],
)
def _sc_edges(xn, ea, src, dst, out,
              src_v0, src_v1, src_v2, src_v3,
              dst_v0, dst_v1, dst_v2, dst_v3,
              rows_v0, rows_v1, msg_v0, msg_v1, zero_v, agg_sh,
              isem0, isem1, isem2, isem3,
              gsem0, gsem1, esem0, esem1, ssem0, ssem1, usem):
    c = lax.axis_index("c")
    s = lax.axis_index("s")
    wid = c * NS + s
    src_vs = [src_v0, src_v1, src_v2, src_v3]
    dst_vs = [dst_v0, dst_v1, dst_v2, dst_v3]
    rows_vs = [rows_v0, rows_v1]
    msg_vs = [msg_v0, msg_v1]
    isems = [isem0, isem1, isem2, isem3]
    gsems = [gsem0, gsem1]
    esems = [esem0, esem1]
    ssems = [ssem0, ssem1]

    def zfill(i, carry):
        for h in range(NSUB):
            zero_v[i, pl.ds(h * 16, 16)] = jnp.zeros((16,), jnp.float32)
        return carry

    lax.fori_loop(0, UR, zfill, 0)
    nunits = (NU - s + NS - 1) // NS

    def zunit(j, carry):
        u = s + NS * j
        pltpu.async_copy(zero_v, agg_sh.at[pl.ds(u * UR, UR), :], usem)
        return carry

    lax.fori_loop(0, nunits, zunit, 0)

    def zdrain(j, carry):
        pltpu.make_async_copy(zero_v, agg_sh.at[pl.ds(0, UR), :], usem).wait()
        return carry

    lax.fori_loop(0, nunits, zdrain, 0)
    plsc.subcore_barrier()

    def fetch_idx(i, ch):
        base = wid * EPW + ch * CH
        pltpu.async_copy(src.at[pl.ds(base, CH)], src_vs[i], isems[i])
        pltpu.async_copy(dst.at[pl.ds(base, CH)], dst_vs[i], isems[i])

    def fetch_data(b, i, ws, ch, first):
        base = wid * EPW + ch * CH
        pltpu.make_async_copy(src.at[pl.ds(0, CH)], src_vs[i],
                              isems[i]).wait()
        pltpu.make_async_copy(dst.at[pl.ds(0, CH)], dst_vs[i],
                              isems[i]).wait()
        if not first:
            pltpu.make_async_copy(msg_vs[b], agg_sh.at[dst_vs[ws]],
                                  ssems[b]).wait()
        pltpu.async_copy(xn.at[src_vs[i]], rows_vs[b], gsems[b])
        pltpu.async_copy(ea.at[pl.ds(base, CH), :], msg_vs[b], esems[b])

    def consume(b, i):
        pltpu.make_async_copy(xn.at[src_vs[i]], rows_vs[b], gsems[b]).wait()
        pltpu.make_async_copy(ea.at[pl.ds(0, CH), :], msg_vs[b],
                              esems[b]).wait()

        def _edge(e):
            for h in range(NSUB):
                sl = pl.ds(h * 16, 16)
                msg_vs[b][e, sl] = jnp.maximum(
                    rows_vs[b][e, sl] + msg_vs[b][e, sl], 0.0)

        plsc.parallel_loop(0, CH, unroll=4)(_edge)

        pltpu.async_copy(msg_vs[b], agg_sh.at[dst_vs[i]], ssems[b], add=True)

    fetch_idx(0, 0)
    fetch_idx(1, 1)
    fetch_idx(2, 2)
    fetch_data(0, 0, 0, 0, True)
    fetch_data(1, 1, 0, 1, True)
    fetch_idx(3, 3)
    consume(0, 0)
    fetch_data(0, 2, 0, 2, False)
    fetch_idx(0, 4)
    consume(1, 1)
    fetch_data(1, 3, 1, 3, False)
    fetch_idx(1, 5)
    consume(0, 2)
    fetch_data(0, 0, 2, 4, False)
    fetch_idx(2, 6)
    consume(1, 3)

    def quad(j, carry):
        for t in range(4):
            k = 4 * j + 4 + t
            fetch_data((t + 1) % 2, (t + 1) % 4, (t + 3) % 4, k + 1, False)
            fetch_idx((t + 3) % 4, jnp.minimum(k + 3, NCHUNK - 1))
            consume(t % 2, t % 4)
        return carry

    lax.fori_loop(0, (NCHUNK - 5) // 4, quad, 0)

    consume(0, 0)
    pltpu.make_async_copy(msg_vs[1], agg_sh.at[dst_vs[3]], ssems[1]).wait()
    pltpu.make_async_copy(msg_vs[0], agg_sh.at[dst_vs[0]], ssems[0]).wait()
    for i in (1, 2):
        pltpu.make_async_copy(src.at[pl.ds(0, CH)], src_vs[i],
                              isems[i]).wait()
        pltpu.make_async_copy(dst.at[pl.ds(0, CH)], dst_vs[i],
                              isems[i]).wait()

    plsc.subcore_barrier()

    def wunit(j, carry):
        u = s + NS * j
        rows = pl.ds(u * UR, UR)
        pltpu.async_copy(agg_sh.at[rows, :], out.at[c, rows, :], usem)
        return carry

    lax.fori_loop(0, nunits, wunit, 0)

    def wdrain(j, carry):
        pltpu.make_async_copy(agg_sh.at[pl.ds(0, UR), :],
                              out.at[c, pl.ds(0, UR), :], usem).wait()
        return carry

    lax.fori_loop(0, nunits, wdrain, 0)


def kernel(x, edge_index, edge_attr, Wn1, We1, Ws1, b1, Wn2, We2, Ws2, b2,
           Wn3, We3, Ws3, b3, Wn4, We4, Ws4, b4):
    src = edge_index[0]
    dst = edge_index[1]
    layers = [(Wn1, We1, Ws1, b1), (Wn2, We2, Ws2, b2),
              (Wn3, We3, Ws3, b3), (Wn4, We4, Ws4, b4)]
    eas = [_matmul(edge_attr, We, 8000) for _, We, _, _ in layers]
    Wn1_, _, Ws1_, b1_ = layers[0]
    xn, hs = _mm2(x, Wn1_, Ws1_, b1_.reshape(1, -1), 2000)
    agg = _sc_edges(xn, eas[0], src, dst)
    for (Wn, _, Ws, b), ea in zip(layers[1:], eas[1:]):
        xn, hs = _fused(hs, agg, Wn, Ws, b.reshape(1, -1), 2000)
        agg = _sc_edges(xn, ea, src, dst)
    return _update(hs, agg, 2000)
